# baseline (speedup 1.0000x reference)
import jax
import jax.numpy as jnp
from jax import lax
from jax.experimental import pallas as pl
from jax.experimental.pallas import tpu as pltpu
from jax.experimental.pallas import tpu_sc as plsc

DIM = 768
HID = 4 * DIM
NE = 64
T = 2048
TM = 32
_RAW_STEPS = T // TM + NE - 1
MAX_STEPS = ((_RAW_STEPS + 7) // 8) * 8
PAD_ROWS = MAX_STEPS * TM
SCW = 128
DIMH = DIM // 2
CH = 128


def _router_body(u_ref, c_ref, b_ref, dest_ref, gate_ref, se_ref, nt_ref,
                 mv_ref, uh_ref):
    x = u_ref[...]
    c = c_ref[...]
    uh_ref[0] = x[:, :DIMH]
    uh_ref[1] = x[:, DIMH:]
    s = lax.dot_general(x, c, (((1,), (1,)), ((), ())),
                        preferred_element_type=jnp.float32)
    s = s + b_ref[...]
    m = jnp.max(s, axis=1, keepdims=True)
    den = jnp.sum(jnp.exp(s - m), axis=1, keepdims=True)
    gate_ref[...] = 1.0 / den

    lane = lax.broadcasted_iota(jnp.int32, (T, NE), 1)
    idx = jnp.min(jnp.where(s == m, lane, NE), axis=1, keepdims=True)
    onehot = (lane == idx).astype(jnp.float32)
    counts = jnp.sum(onehot, axis=0, keepdims=True)
    perfect = float(T // NE)
    maxf = jnp.max(counts, axis=1, keepdims=True)
    mv_ref[...] = (maxf - perfect) / perfect

    cnt_i = counts.astype(jnp.int32)
    tiles_f = ((cnt_i + (TM - 1)) // TM).astype(jnp.float32)
    r64 = lax.broadcasted_iota(jnp.int32, (NE, NE), 0)
    c64 = lax.broadcasted_iota(jnp.int32, (NE, NE), 1)
    tri_incl = (r64 <= c64).astype(jnp.float32)
    cum_incl = lax.dot_general(tiles_f, tri_incl, (((1,), (0,)), ((), ())),
                               preferred_element_type=jnp.float32)
    padded_off = (cum_incl - tiles_f) * TM

    rl = lax.broadcasted_iota(jnp.int32, (CH, CH), 0)
    cl = lax.broadcasted_iota(jnp.int32, (CH, CH), 1)
    ltri = (rl >= cl).astype(jnp.float32)
    base = jnp.zeros((1, NE), jnp.float32)
    chunks = []
    for k in range(T // CH):
        oh = onehot[k * CH:(k + 1) * CH]
        run = lax.dot_general(ltri, oh, (((1,), (0,)), ((), ())),
                              preferred_element_type=jnp.float32)
        coef = run - 1.0 + base + padded_off
        chunks.append(jnp.sum(coef * oh, axis=1, keepdims=True))
        base = base + jnp.sum(oh, axis=0, keepdims=True)
    dest = jnp.concatenate(chunks, axis=0).astype(jnp.int32)
    dest_ref[...] = jnp.concatenate([dest, dest + PAD_ROWS], axis=1)

    cum_i = cum_incl.astype(jnp.int32)
    ntile = cum_i[:, NE - 1:]
    srow = lax.broadcasted_iota(jnp.int32, (MAX_STEPS, NE), 0)
    se = jnp.sum((cum_i <= srow).astype(jnp.int32), axis=1, keepdims=True)
    lastv = jnp.sum((cum_i <= ntile - 1).astype(jnp.int32), axis=1,
                    keepdims=True)
    se_ref[...] = jnp.minimum(se, lastv)
    nt_ref[...] = ntile


def _mlp_body(se_ref, nt_ref, xa_ref, xb_ref, w1a_ref, w1b_ref, b1_ref,
              w2a_ref, w2b_ref, b2_ref, o_ref):
    s = pl.program_id(0)

    @pl.when(s < nt_ref[0])
    def _():
        x = jnp.concatenate([xa_ref[...], xb_ref[...]],
                            axis=1).astype(jnp.bfloat16)
        ha = lax.dot_general(x, w1a_ref[0].astype(jnp.bfloat16),
                             (((1,), (1,)), ((), ())),
                             preferred_element_type=jnp.float32)
        hb = lax.dot_general(x, w1b_ref[0].astype(jnp.bfloat16),
                             (((1,), (1,)), ((), ())),
                             preferred_element_type=jnp.float32)
        h = jnp.concatenate([ha, hb], axis=1)
        h = jnp.maximum(h + b1_ref[0], 0.0).astype(jnp.bfloat16)
        y = lax.dot_general(h[:, :HID // 2], w2a_ref[0].astype(jnp.bfloat16),
                            (((1,), (1,)), ((), ())),
                            preferred_element_type=jnp.float32)
        y = y + lax.dot_general(h[:, HID // 2:],
                                w2b_ref[0].astype(jnp.bfloat16),
                                (((1,), (1,)), ((), ())),
                                preferred_element_type=jnp.float32)
        y = y + b2_ref[0]
        o_ref[0] = y[:, :DIMH]
        o_ref[1] = y[:, DIMH:]


def _scale_body(ya_ref, yb_ref, g_ref, o_ref):
    g = g_ref[...]
    o_ref[...] = jnp.concatenate([ya_ref[...] * g, yb_ref[...] * g], axis=1)


def _router(u_flat, centroids, biases_row):
    return pl.pallas_call(
        _router_body,
        out_shape=(
            jax.ShapeDtypeStruct((T, 2), jnp.int32),
            jax.ShapeDtypeStruct((T, 1), jnp.float32),
            jax.ShapeDtypeStruct((MAX_STEPS, 1), jnp.int32),
            jax.ShapeDtypeStruct((1, 1), jnp.int32),
            jax.ShapeDtypeStruct((1, 1), jnp.float32),
            jax.ShapeDtypeStruct((2, T, DIMH), jnp.float32),
        ),
    )(u_flat, centroids, biases_row)


def _sc_scatter(u_half, dest_row):
    mesh = plsc.VectorSubcoreMesh(core_axis_name="c", subcore_axis_name="s")
    nwin = T // SCW

    @pl.kernel(out_type=jax.ShapeDtypeStruct((2 * PAD_ROWS, DIMH),
                                             jnp.float32),
               mesh=mesh)
    def k(x_hbm, i_hbm, o_hbm):
        def body(x_vmem, i_vmem):
            pltpu.sync_copy(x_vmem, o_hbm.at[i_vmem.at[0]])

        pltpu.emit_pipeline(
            body,
            grid=(2 * nwin,),
            in_specs=[pl.BlockSpec((SCW, DIMH), lambda i: (i, 0)),
                      pl.BlockSpec((1, SCW), lambda i: (i // nwin, i % nwin))],
            out_specs=[],
            core_axis_name=("c", "s"),
            dimension_semantics=(pltpu.PARALLEL,),
        )(x_hbm, i_hbm)

    return k(u_half, dest_row)


def _sc_gather(y_half, dest_row):
    mesh = plsc.VectorSubcoreMesh(core_axis_name="c", subcore_axis_name="s")
    nwin = T // SCW

    @pl.kernel(out_type=jax.ShapeDtypeStruct((2 * T, DIMH), jnp.float32),
               mesh=mesh)
    def k(y_hbm, i_hbm, o_hbm):
        def body(i_vmem, o_vmem):
            pltpu.sync_copy(y_hbm.at[i_vmem.at[0]], o_vmem)

        pltpu.emit_pipeline(
            body,
            grid=(2 * nwin,),
            in_specs=[pl.BlockSpec((1, SCW), lambda i: (i // nwin, i % nwin))],
            out_specs=[pl.BlockSpec((SCW, DIMH), lambda i: (i, 0))],
            core_axis_name=("c", "s"),
            dimension_semantics=(pltpu.PARALLEL,),
        )(i_hbm, o_hbm)

    return k(y_half, dest_row)


def _mlp(se, nt, u_pad, W1, b1, W2, b2):
    grid_spec = pltpu.PrefetchScalarGridSpec(
        num_scalar_prefetch=2,
        grid=(MAX_STEPS,),
        in_specs=[
            pl.BlockSpec((TM, DIMH),
                         lambda s, se, nt: (jnp.minimum(s, nt[0] - 1), 0)),
            pl.BlockSpec((TM, DIMH),
                         lambda s, se, nt: (MAX_STEPS
                                            + jnp.minimum(s, nt[0] - 1), 0)),
            pl.BlockSpec((1, HID // 2, DIM), lambda s, se, nt: (se[s], 0, 0)),
            pl.BlockSpec((1, HID // 2, DIM), lambda s, se, nt: (se[s], 1, 0)),
            pl.BlockSpec((1, 1, HID), lambda s, se, nt: (se[s], 0, 0)),
            pl.BlockSpec((1, DIM, HID // 2), lambda s, se, nt: (se[s], 0, 0)),
            pl.BlockSpec((1, DIM, HID // 2), lambda s, se, nt: (se[s], 0, 1)),
            pl.BlockSpec((1, 1, DIM), lambda s, se, nt: (se[s], 0, 0)),
        ],
        out_specs=pl.BlockSpec(
            (2, TM, DIMH),
            lambda s, se, nt: (0, jnp.minimum(s, nt[0] - 1), 0)),
    )
    return pl.pallas_call(
        _mlp_body,
        grid_spec=grid_spec,
        out_shape=jax.ShapeDtypeStruct((2, PAD_ROWS, DIMH), jnp.float32),
    )(se, nt, u_pad, u_pad, W1, W1, b1.reshape(NE, 1, HID), W2, W2,
      b2.reshape(NE, 1, DIM))


def _scale(y_tok2, gate):
    return pl.pallas_call(
        _scale_body,
        grid=(1,),
        in_specs=[
            pl.BlockSpec((T, DIMH), lambda i: (0, 0)),
            pl.BlockSpec((T, DIMH), lambda i: (1, 0)),
            pl.BlockSpec((T, 1), lambda i: (0, 0)),
        ],
        out_specs=pl.BlockSpec((T, DIM), lambda i: (0, 0)),
        out_shape=jax.ShapeDtypeStruct((T, DIM), jnp.float32),
    )(y_tok2, y_tok2, gate)


def kernel(u_t, centroids, score_biases, W1, b1, W2, b2):
    Bb, Ss, dim = u_t.shape
    u_flat = u_t.reshape(Bb * Ss, dim)
    dest2, gate, se, nt, mv, u_half3 = _router(u_flat, centroids,
                                               score_biases.reshape(1, NE))
    dest_row = dest2.T
    u_pad = _sc_scatter(u_half3.reshape(2 * T, DIMH), dest_row)
    y_pad = _mlp(se.reshape(MAX_STEPS), nt.reshape(1), u_pad, W1, b1, W2, b2)
    y_tok2 = _sc_gather(y_pad.reshape(2 * PAD_ROWS, DIMH), dest_row)
    out = _scale(y_tok2, gate)
    return out.reshape(Bb, Ss, dim), mv.reshape(())


# speedup vs baseline: 1.3084x; 1.3084x over previous
import jax
import jax.numpy as jnp
from jax import lax
from jax.experimental import pallas as pl
from jax.experimental.pallas import tpu as pltpu
from jax.experimental.pallas import tpu_sc as plsc

DIM = 768
HID = 4 * DIM
NE = 64
T = 2048
TM = 64
_RAW_STEPS = T // TM + NE - 1
MAX_STEPS = ((_RAW_STEPS + 7) // 8) * 8
PAD_ROWS = MAX_STEPS * TM
SCW = 128
DIMH = DIM // 2
CH = 128


def _router_body(u_ref, c_ref, b_ref, dest_ref, gate_ref, se_ref, nt_ref,
                 mv_ref, uh_ref):
    x = u_ref[...]
    c = c_ref[...]
    uh_ref[0] = x[:, :DIMH]
    uh_ref[1] = x[:, DIMH:]
    s = lax.dot_general(x, c, (((1,), (1,)), ((), ())),
                        preferred_element_type=jnp.float32)
    s = s + b_ref[...]
    m = jnp.max(s, axis=1, keepdims=True)
    den = jnp.sum(jnp.exp(s - m), axis=1, keepdims=True)
    gate_ref[...] = 1.0 / den

    lane = lax.broadcasted_iota(jnp.int32, (T, NE), 1)
    idx = jnp.min(jnp.where(s == m, lane, NE), axis=1, keepdims=True)
    onehot = (lane == idx).astype(jnp.float32)
    counts = jnp.sum(onehot, axis=0, keepdims=True)
    perfect = float(T // NE)
    maxf = jnp.max(counts, axis=1, keepdims=True)
    mv_ref[...] = (maxf - perfect) / perfect

    cnt_i = counts.astype(jnp.int32)
    tiles_f = ((cnt_i + (TM - 1)) // TM).astype(jnp.float32)
    r64 = lax.broadcasted_iota(jnp.int32, (NE, NE), 0)
    c64 = lax.broadcasted_iota(jnp.int32, (NE, NE), 1)
    tri_incl = (r64 <= c64).astype(jnp.float32)
    cum_incl = lax.dot_general(tiles_f, tri_incl, (((1,), (0,)), ((), ())),
                               preferred_element_type=jnp.float32)
    padded_off = (cum_incl - tiles_f) * TM

    rl = lax.broadcasted_iota(jnp.int32, (CH, CH), 0)
    cl = lax.broadcasted_iota(jnp.int32, (CH, CH), 1)
    ltri = (rl >= cl).astype(jnp.float32)
    base = jnp.zeros((1, NE), jnp.float32)
    chunks = []
    for k in range(T // CH):
        oh = onehot[k * CH:(k + 1) * CH]
        run = lax.dot_general(ltri, oh, (((1,), (0,)), ((), ())),
                              preferred_element_type=jnp.float32)
        coef = run - 1.0 + base + padded_off
        chunks.append(jnp.sum(coef * oh, axis=1, keepdims=True))
        base = base + jnp.sum(oh, axis=0, keepdims=True)
    dest = jnp.concatenate(chunks, axis=0).astype(jnp.int32)
    dest_ref[...] = jnp.concatenate([dest, dest + PAD_ROWS], axis=1)

    cum_i = cum_incl.astype(jnp.int32)
    ntile = cum_i[:, NE - 1:]
    srow = lax.broadcasted_iota(jnp.int32, (MAX_STEPS, NE), 0)
    se = jnp.sum((cum_i <= srow).astype(jnp.int32), axis=1, keepdims=True)
    lastv = jnp.sum((cum_i <= ntile - 1).astype(jnp.int32), axis=1,
                    keepdims=True)
    se_ref[...] = jnp.minimum(se, lastv)
    nt_ref[...] = ntile


def _mlp_body(se_ref, nt_ref, xa_ref, xb_ref, w1a_ref, w1b_ref, b1_ref,
              w2a_ref, w2b_ref, b2_ref, o_ref):
    s = pl.program_id(0)

    @pl.when(s < nt_ref[0])
    def _():
        x = jnp.concatenate([xa_ref[...], xb_ref[...]],
                            axis=1).astype(jnp.bfloat16)
        ha = lax.dot_general(x, w1a_ref[0].astype(jnp.bfloat16),
                             (((1,), (1,)), ((), ())),
                             preferred_element_type=jnp.float32)
        hb = lax.dot_general(x, w1b_ref[0].astype(jnp.bfloat16),
                             (((1,), (1,)), ((), ())),
                             preferred_element_type=jnp.float32)
        h = jnp.concatenate([ha, hb], axis=1)
        h = jnp.maximum(h + b1_ref[0], 0.0).astype(jnp.bfloat16)
        y = lax.dot_general(h[:, :HID // 2], w2a_ref[0].astype(jnp.bfloat16),
                            (((1,), (1,)), ((), ())),
                            preferred_element_type=jnp.float32)
        y = y + lax.dot_general(h[:, HID // 2:],
                                w2b_ref[0].astype(jnp.bfloat16),
                                (((1,), (1,)), ((), ())),
                                preferred_element_type=jnp.float32)
        y = y + b2_ref[0]
        o_ref[0] = y[:, :DIMH]
        o_ref[1] = y[:, DIMH:]


def _scale_body(ya_ref, yb_ref, g_ref, o_ref):
    g = g_ref[...]
    o_ref[...] = jnp.concatenate([ya_ref[...] * g, yb_ref[...] * g], axis=1)


def _router(u_flat, centroids, biases_row):
    return pl.pallas_call(
        _router_body,
        out_shape=(
            jax.ShapeDtypeStruct((T, 2), jnp.int32),
            jax.ShapeDtypeStruct((T, 1), jnp.float32),
            jax.ShapeDtypeStruct((MAX_STEPS, 1), jnp.int32),
            jax.ShapeDtypeStruct((1, 1), jnp.int32),
            jax.ShapeDtypeStruct((1, 1), jnp.float32),
            jax.ShapeDtypeStruct((2, T, DIMH), jnp.float32),
        ),
    )(u_flat, centroids, biases_row)


def _sc_scatter(u_half, dest_row):
    mesh = plsc.VectorSubcoreMesh(core_axis_name="c", subcore_axis_name="s")
    nwin = T // SCW

    @pl.kernel(out_type=jax.ShapeDtypeStruct((2 * PAD_ROWS, DIMH),
                                             jnp.float32),
               mesh=mesh)
    def k(x_hbm, i_hbm, o_hbm):
        def body(x_vmem, i_vmem):
            pltpu.sync_copy(x_vmem, o_hbm.at[i_vmem.at[0]])

        pltpu.emit_pipeline(
            body,
            grid=(2 * nwin,),
            in_specs=[pl.BlockSpec((SCW, DIMH), lambda i: (i, 0)),
                      pl.BlockSpec((1, SCW), lambda i: (i // nwin, i % nwin))],
            out_specs=[],
            core_axis_name=("c", "s"),
            dimension_semantics=(pltpu.PARALLEL,),
        )(x_hbm, i_hbm)

    return k(u_half, dest_row)


def _sc_gather(y_half, dest_row):
    mesh = plsc.VectorSubcoreMesh(core_axis_name="c", subcore_axis_name="s")
    nwin = T // SCW

    @pl.kernel(out_type=jax.ShapeDtypeStruct((2 * T, DIMH), jnp.float32),
               mesh=mesh)
    def k(y_hbm, i_hbm, o_hbm):
        def body(i_vmem, o_vmem):
            pltpu.sync_copy(y_hbm.at[i_vmem.at[0]], o_vmem)

        pltpu.emit_pipeline(
            body,
            grid=(2 * nwin,),
            in_specs=[pl.BlockSpec((1, SCW), lambda i: (i // nwin, i % nwin))],
            out_specs=[pl.BlockSpec((SCW, DIMH), lambda i: (i, 0))],
            core_axis_name=("c", "s"),
            dimension_semantics=(pltpu.PARALLEL,),
        )(i_hbm, o_hbm)

    return k(y_half, dest_row)


def _mlp(se, nt, u_pad, W1, b1, W2, b2):
    grid_spec = pltpu.PrefetchScalarGridSpec(
        num_scalar_prefetch=2,
        grid=(MAX_STEPS,),
        in_specs=[
            pl.BlockSpec((TM, DIMH),
                         lambda s, se, nt: (jnp.minimum(s, nt[0] - 1), 0)),
            pl.BlockSpec((TM, DIMH),
                         lambda s, se, nt: (MAX_STEPS
                                            + jnp.minimum(s, nt[0] - 1), 0)),
            pl.BlockSpec((1, HID // 2, DIM), lambda s, se, nt: (se[s], 0, 0)),
            pl.BlockSpec((1, HID // 2, DIM), lambda s, se, nt: (se[s], 1, 0)),
            pl.BlockSpec((1, 1, HID), lambda s, se, nt: (se[s], 0, 0)),
            pl.BlockSpec((1, DIM, HID // 2), lambda s, se, nt: (se[s], 0, 0)),
            pl.BlockSpec((1, DIM, HID // 2), lambda s, se, nt: (se[s], 0, 1)),
            pl.BlockSpec((1, 1, DIM), lambda s, se, nt: (se[s], 0, 0)),
        ],
        out_specs=pl.BlockSpec(
            (2, TM, DIMH),
            lambda s, se, nt: (0, jnp.minimum(s, nt[0] - 1), 0)),
    )
    return pl.pallas_call(
        _mlp_body,
        grid_spec=grid_spec,
        out_shape=jax.ShapeDtypeStruct((2, PAD_ROWS, DIMH), jnp.float32),
    )(se, nt, u_pad, u_pad, W1, W1, b1.reshape(NE, 1, HID), W2, W2,
      b2.reshape(NE, 1, DIM))


def _scale(y_tok2, gate):
    return pl.pallas_call(
        _scale_body,
        grid=(1,),
        in_specs=[
            pl.BlockSpec((T, DIMH), lambda i: (0, 0)),
            pl.BlockSpec((T, DIMH), lambda i: (1, 0)),
            pl.BlockSpec((T, 1), lambda i: (0, 0)),
        ],
        out_specs=pl.BlockSpec((T, DIM), lambda i: (0, 0)),
        out_shape=jax.ShapeDtypeStruct((T, DIM), jnp.float32),
    )(y_tok2, y_tok2, gate)


def kernel(u_t, centroids, score_biases, W1, b1, W2, b2):
    Bb, Ss, dim = u_t.shape
    u_flat = u_t.reshape(Bb * Ss, dim)
    dest2, gate, se, nt, mv, u_half3 = _router(u_flat, centroids,
                                               score_biases.reshape(1, NE))
    dest_row = dest2.T
    u_pad = _sc_scatter(u_half3.reshape(2 * T, DIMH), dest_row)
    y_pad = _mlp(se.reshape(MAX_STEPS), nt.reshape(1), u_pad, W1, b1, W2, b2)
    y_tok2 = _sc_gather(y_pad.reshape(2 * PAD_ROWS, DIMH), dest_row)
    out = _scale(y_tok2, gate)
    return out.reshape(Bb, Ss, dim), mv.reshape(())


# speedup vs baseline: 1.3189x; 1.0080x over previous
import jax
import jax.numpy as jnp
from jax import lax
from jax.experimental import pallas as pl
from jax.experimental.pallas import tpu as pltpu
from jax.experimental.pallas import tpu_sc as plsc

DIM = 768
HID = 4 * DIM
NE = 64
T = 2048
TM = 64
_RAW_STEPS = T // TM + NE - 1
MAX_STEPS = ((_RAW_STEPS + 7) // 8) * 8
PAD_ROWS = MAX_STEPS * TM
SCW = 128
DIMH = DIM // 2
CH = 128


def _router_body(u_ref, c_ref, b_ref, dest_ref, gate_ref, se_ref, nt_ref,
                 mv_ref, uh_ref):
    x = u_ref[...]
    c = c_ref[...]
    uh_ref[0] = x[:, :DIMH]
    uh_ref[1] = x[:, DIMH:]
    s = lax.dot_general(x, c, (((1,), (1,)), ((), ())),
                        preferred_element_type=jnp.float32)
    s = s + b_ref[...]
    m = jnp.max(s, axis=1, keepdims=True)
    den = jnp.sum(jnp.exp(s - m), axis=1, keepdims=True)
    gate_ref[...] = 1.0 / den

    lane = lax.broadcasted_iota(jnp.int32, (T, NE), 1)
    idx = jnp.min(jnp.where(s == m, lane, NE), axis=1, keepdims=True)
    onehot = (lane == idx).astype(jnp.float32)
    counts = jnp.sum(onehot, axis=0, keepdims=True)
    perfect = float(T // NE)
    maxf = jnp.max(counts, axis=1, keepdims=True)
    mv_ref[...] = (maxf - perfect) / perfect

    cnt_i = counts.astype(jnp.int32)
    tiles_f = ((cnt_i + (TM - 1)) // TM).astype(jnp.float32)
    r64 = lax.broadcasted_iota(jnp.int32, (NE, NE), 0)
    c64 = lax.broadcasted_iota(jnp.int32, (NE, NE), 1)
    tri_incl = (r64 <= c64).astype(jnp.float32)
    cum_incl = lax.dot_general(tiles_f, tri_incl, (((1,), (0,)), ((), ())),
                               preferred_element_type=jnp.float32)
    padded_off = (cum_incl - tiles_f) * TM

    rl = lax.broadcasted_iota(jnp.int32, (CH, CH), 0)
    cl = lax.broadcasted_iota(jnp.int32, (CH, CH), 1)
    ltri = (rl >= cl).astype(jnp.float32)
    base = jnp.zeros((1, NE), jnp.float32)
    chunks = []
    for k in range(T // CH):
        oh = onehot[k * CH:(k + 1) * CH]
        run = lax.dot_general(ltri, oh, (((1,), (0,)), ((), ())),
                              preferred_element_type=jnp.float32)
        coef = run - 1.0 + base + padded_off
        chunks.append(jnp.sum(coef * oh, axis=1, keepdims=True))
        base = base + jnp.sum(oh, axis=0, keepdims=True)
    dest = jnp.concatenate(chunks, axis=0).astype(jnp.int32)
    drow = jnp.transpose(dest)
    dest_ref[...] = jnp.concatenate([drow, drow + PAD_ROWS], axis=0)

    cum_i = cum_incl.astype(jnp.int32)
    ntile = cum_i[:, NE - 1:]
    srow = lax.broadcasted_iota(jnp.int32, (MAX_STEPS, NE), 0)
    se = jnp.sum((cum_i <= srow).astype(jnp.int32), axis=1, keepdims=True)
    lastv = jnp.sum((cum_i <= ntile - 1).astype(jnp.int32), axis=1,
                    keepdims=True)
    se_ref[...] = jnp.minimum(se, lastv)
    nt_ref[...] = ntile


def _mlp_body(se_ref, nt_ref, xa_ref, xb_ref, w1a_ref, w1b_ref, b1_ref,
              w2a_ref, w2b_ref, b2_ref, o_ref):
    s = pl.program_id(0)

    @pl.when(s < nt_ref[0, 0])
    def _():
        x = jnp.concatenate([xa_ref[...], xb_ref[...]],
                            axis=1).astype(jnp.bfloat16)
        ha = lax.dot_general(x, w1a_ref[0].astype(jnp.bfloat16),
                             (((1,), (1,)), ((), ())),
                             preferred_element_type=jnp.float32)
        hb = lax.dot_general(x, w1b_ref[0].astype(jnp.bfloat16),
                             (((1,), (1,)), ((), ())),
                             preferred_element_type=jnp.float32)
        h = jnp.concatenate([ha, hb], axis=1)
        h = jnp.maximum(h + b1_ref[0], 0.0).astype(jnp.bfloat16)
        y = lax.dot_general(h[:, :HID // 2], w2a_ref[0].astype(jnp.bfloat16),
                            (((1,), (1,)), ((), ())),
                            preferred_element_type=jnp.float32)
        y = y + lax.dot_general(h[:, HID // 2:],
                                w2b_ref[0].astype(jnp.bfloat16),
                                (((1,), (1,)), ((), ())),
                                preferred_element_type=jnp.float32)
        y = y + b2_ref[0]
        o_ref[0] = y[:, :DIMH]
        o_ref[1] = y[:, DIMH:]


def _scale_body(ya_ref, yb_ref, g_ref, o_ref):
    g = g_ref[...]
    o_ref[...] = jnp.concatenate([ya_ref[...] * g, yb_ref[...] * g], axis=1)


def _router(u_flat, centroids, biases_row):
    return pl.pallas_call(
        _router_body,
        out_shape=(
            jax.ShapeDtypeStruct((2, T), jnp.int32),
            jax.ShapeDtypeStruct((T, 1), jnp.float32),
            jax.ShapeDtypeStruct((MAX_STEPS, 1), jnp.int32),
            jax.ShapeDtypeStruct((1, 1), jnp.int32),
            jax.ShapeDtypeStruct((1, 1), jnp.float32),
            jax.ShapeDtypeStruct((2, T, DIMH), jnp.float32),
        ),
    )(u_flat, centroids, biases_row)


def _sc_scatter(u_half, dest_row):
    mesh = plsc.VectorSubcoreMesh(core_axis_name="c", subcore_axis_name="s")
    nwin = T // SCW

    @pl.kernel(out_type=jax.ShapeDtypeStruct((2 * PAD_ROWS, DIMH),
                                             jnp.float32),
               mesh=mesh)
    def k(x_hbm, i_hbm, o_hbm):
        def body(x_vmem, i_vmem):
            pltpu.sync_copy(x_vmem, o_hbm.at[i_vmem.at[0]])

        pltpu.emit_pipeline(
            body,
            grid=(2 * nwin,),
            in_specs=[pl.BlockSpec((SCW, DIMH), lambda i: (i, 0)),
                      pl.BlockSpec((1, SCW), lambda i: (i // nwin, i % nwin))],
            out_specs=[],
            core_axis_name=("c", "s"),
            dimension_semantics=(pltpu.PARALLEL,),
        )(x_hbm, i_hbm)

    return k(u_half, dest_row)


def _sc_gather(y_half, dest_row):
    mesh = plsc.VectorSubcoreMesh(core_axis_name="c", subcore_axis_name="s")
    nwin = T // SCW

    @pl.kernel(out_type=jax.ShapeDtypeStruct((2 * T, DIMH), jnp.float32),
               mesh=mesh)
    def k(y_hbm, i_hbm, o_hbm):
        def body(i_vmem, o_vmem):
            pltpu.sync_copy(y_hbm.at[i_vmem.at[0]], o_vmem)

        pltpu.emit_pipeline(
            body,
            grid=(2 * nwin,),
            in_specs=[pl.BlockSpec((1, SCW), lambda i: (i // nwin, i % nwin))],
            out_specs=[pl.BlockSpec((SCW, DIMH), lambda i: (i, 0))],
            core_axis_name=("c", "s"),
            dimension_semantics=(pltpu.PARALLEL,),
        )(i_hbm, o_hbm)

    return k(y_half, dest_row)


def _mlp(se, nt, u_pad, W1, b1, W2, b2):
    grid_spec = pltpu.PrefetchScalarGridSpec(
        num_scalar_prefetch=2,
        grid=(MAX_STEPS,),
        in_specs=[
            pl.BlockSpec((TM, DIMH),
                         lambda s, se, nt: (jnp.minimum(s, nt[0, 0] - 1), 0)),
            pl.BlockSpec((TM, DIMH),
                         lambda s, se, nt: (MAX_STEPS
                                            + jnp.minimum(s, nt[0, 0] - 1), 0)),
            pl.BlockSpec((1, HID // 2, DIM), lambda s, se, nt: (se[s, 0], 0, 0)),
            pl.BlockSpec((1, HID // 2, DIM), lambda s, se, nt: (se[s, 0], 1, 0)),
            pl.BlockSpec((1, 1, HID), lambda s, se, nt: (se[s, 0], 0, 0)),
            pl.BlockSpec((1, DIM, HID // 2), lambda s, se, nt: (se[s, 0], 0, 0)),
            pl.BlockSpec((1, DIM, HID // 2), lambda s, se, nt: (se[s, 0], 0, 1)),
            pl.BlockSpec((1, 1, DIM), lambda s, se, nt: (se[s, 0], 0, 0)),
        ],
        out_specs=pl.BlockSpec(
            (2, TM, DIMH),
            lambda s, se, nt: (0, jnp.minimum(s, nt[0, 0] - 1), 0)),
    )
    return pl.pallas_call(
        _mlp_body,
        grid_spec=grid_spec,
        out_shape=jax.ShapeDtypeStruct((2, PAD_ROWS, DIMH), jnp.float32),
    )(se, nt, u_pad, u_pad, W1, W1, b1.reshape(NE, 1, HID), W2, W2,
      b2.reshape(NE, 1, DIM))


def _scale(y_tok2, gate):
    return pl.pallas_call(
        _scale_body,
        grid=(1,),
        in_specs=[
            pl.BlockSpec((T, DIMH), lambda i: (0, 0)),
            pl.BlockSpec((T, DIMH), lambda i: (1, 0)),
            pl.BlockSpec((T, 1), lambda i: (0, 0)),
        ],
        out_specs=pl.BlockSpec((T, DIM), lambda i: (0, 0)),
        out_shape=jax.ShapeDtypeStruct((T, DIM), jnp.float32),
    )(y_tok2, y_tok2, gate)


def kernel(u_t, centroids, score_biases, W1, b1, W2, b2):
    Bb, Ss, dim = u_t.shape
    u_flat = u_t.reshape(Bb * Ss, dim)
    dest_row, gate, se, nt, mv, u_half3 = _router(u_flat, centroids,
                                                  score_biases.reshape(1, NE))
    u_pad = _sc_scatter(u_half3.reshape(2 * T, DIMH), dest_row)
    y_pad = _mlp(se, nt, u_pad, W1, b1, W2, b2)
    y_tok2 = _sc_gather(y_pad.reshape(2 * PAD_ROWS, DIMH), dest_row)
    out = _scale(y_tok2, gate)
    return out.reshape(Bb, Ss, dim), mv.reshape(())
